# baseline (speedup 1.0000x reference)
import functools

import jax
import jax.numpy as jnp
from jax import lax
from jax.experimental import pallas as pl
from jax.experimental.pallas import tpu as pltpu
from jax.experimental.pallas import tpu_sc as plsc

NPTS = 500
PAD = 512
CF = 160
DIM = 64
NS = 16
PPT = PAD // NS

@functools.lru_cache(maxsize=None)
def _sc_kernels():
    mesh = plsc.VectorSubcoreMesh(
        core_axis_name="c", subcore_axis_name="s", num_cores=1,
        num_subcores=NS)
    params = pltpu.CompilerParams(use_tc_tiling_on_sc=False)
    sc1 = pl.kernel(
        _sc1_body,
        out_type=jax.ShapeDtypeStruct((PAD, DIM), jnp.float32),
        mesh=mesh,
        compiler_params=params,
        scratch_types=[
            pltpu.VMEM((PPT * 8,), jnp.int32),
            pltpu.VMEM((PPT * 4,), jnp.int32),
            pltpu.VMEM((PPT * 8,), jnp.float32),
            pltpu.VMEM((PPT, DIM), jnp.float32),
            pltpu.VMEM((128, DIM), jnp.float32),
            pltpu.VMEM((PPT, DIM), jnp.float32),
            pltpu.VMEM((PPT, DIM), jnp.float32),
            pltpu.VMEM_SHARED((PAD, DIM), jnp.float32),
            pltpu.SemaphoreType.DMA,
        ],
    )
    sc2 = pl.kernel(
        _sc2_body,
        out_type=jax.ShapeDtypeStruct((PAD, CF), jnp.float32),
        mesh=mesh,
        compiler_params=params,
        scratch_types=[
            pltpu.VMEM((PPT * 8,), jnp.int32),
            pltpu.VMEM((PPT * 4,), jnp.int32),
            pltpu.VMEM((128, CF), jnp.float32),
            pltpu.VMEM((PPT, CF), jnp.float32),
            pltpu.VMEM((PPT, CF), jnp.float32),
            pltpu.VMEM((PPT, CF), jnp.float32),
            pltpu.VMEM_SHARED((PAD, CF), jnp.float32),
            pltpu.SemaphoreType.DMA,
        ],
    )
    return sc1, sc2


def _sc1_body(ft, f1t, f2t, i8, i4, wv, plus_out,
              idx8_v, idx4_v, wv_v, ftl, rows, e_acc, h_acc, hsh, sem):
    wid = lax.axis_index("s")
    base = wid * PPT
    pltpu.sync_copy(i8.at[pl.ds(base * 8, PPT * 8)], idx8_v)
    pltpu.sync_copy(wv.at[pl.ds(base * 8, PPT * 8)], wv_v)
    pltpu.sync_copy(i4.at[pl.ds(base * 4, PPT * 4)], idx4_v)
    pltpu.sync_copy(ft.at[pl.ds(base, PPT)], ftl)

    def diff_stage(table, acc):
        for ch in range(2):
            pltpu.async_copy(
                table.at[idx8_v.at[pl.ds(ch * 128, 128)]], rows, sem).wait()

            def acc_body(q, carry):
                wvec = wv_v[pl.ds(ch * 128 + q * 16, 16)] * 0.125
                for t in range(2):
                    p = q * 2 + t
                    g = ch * 16 + p
                    ws = [wvec[t * 8 + j] for j in range(8)]
                    s1 = (ws[0] + ws[1] + ws[2] + ws[3]
                          + ws[4] + ws[5] + ws[6] + ws[7])
                    for c in range(DIM // 16):
                        sl = pl.ds(c * 16, 16)
                        v = ftl[g, sl] * s1
                        for j in range(8):
                            v = v - ws[j] * rows[p * 8 + j, sl]
                        acc[g, sl] = v
                return carry

            lax.fori_loop(0, 8, acc_body, 0)

    diff_stage(f1t, e_acc)
    diff_stage(f2t, h_acc)

    pltpu.sync_copy(h_acc, hsh.at[pl.ds(base, PPT)])
    plsc.subcore_barrier()
    pltpu.async_copy(hsh.at[idx4_v], rows.at[pl.ds(0, 128)], sem).wait()

    def plus_body(p, carry):
        for c in range(DIM // 16):
            sl = pl.ds(c * 16, 16)
            s = rows[4 * p, sl] + rows[4 * p + 1, sl]
            s = s + rows[4 * p + 2, sl] + rows[4 * p + 3, sl]
            e_acc[p, sl] = e_acc[p, sl] + s * 0.25
        return carry

    lax.fori_loop(0, PPT, plus_body, 0)
    pltpu.sync_copy(e_acc, plus_out.at[pl.ds(base, PPT)])


def _sc2_body(dst, dfft, dfst, i2, i14, plus2_out,
              idx8_v, idx4_v, rows, m_acc, dffl, dfsl, nfsh, sem):
    wid = lax.axis_index("s")
    base = wid * PPT
    pltpu.sync_copy(i2.at[pl.ds(base * 8, PPT * 8)], idx8_v)
    pltpu.sync_copy(i14.at[pl.ds(base * 4, PPT * 4)], idx4_v)
    pltpu.sync_copy(dfft.at[pl.ds(base, PPT)], dffl)
    pltpu.sync_copy(dfst.at[pl.ds(base, PPT)], dfsl)

    for ch in range(2):
        pltpu.async_copy(
            dst.at[idx8_v.at[pl.ds(ch * 128, 128)]], rows, sem).wait()

        def times_body(p, carry):
            g = ch * 16 + p
            for c in range(CF // 16):
                sl = pl.ds(c * 16, 16)
                s = rows[p * 8, sl] + rows[p * 8 + 1, sl]
                for j in range(2, 8):
                    s = s + rows[p * 8 + j, sl]
                m_acc[g, sl] = dffl[g, sl] * s * 0.125
            return carry

        lax.fori_loop(0, 16, times_body, 0)

    pltpu.sync_copy(m_acc, nfsh.at[pl.ds(base, PPT)])
    plsc.subcore_barrier()
    pltpu.async_copy(nfsh.at[idx4_v], rows.at[pl.ds(0, 128)], sem).wait()

    def plus_body(p, carry):
        for c in range(CF // 16):
            sl = pl.ds(c * 16, 16)
            s = rows[4 * p, sl] + rows[4 * p + 1, sl]
            s = s + rows[4 * p + 2, sl] + rows[4 * p + 3, sl]
            dffl[p, sl] = dfsl[p, sl] + s * 0.25
        return carry

    lax.fori_loop(0, PPT, plus_body, 0)
    pltpu.sync_copy(dffl, plus2_out.at[pl.ds(base, PPT)])


def _tc_conv1_body(plus_ref, w1t_ref, b1_ref, out_ref):
    out_ref[...] = jax.nn.sigmoid(
        jnp.dot(plus_ref[...], w1t_ref[...],
                preferred_element_type=jnp.float32) + b1_ref[...])


def _tc_conv2_body(plus2_ref, w3_ref, b3_ref, out_ref):
    out_ref[...] = lax.dot_general(
        w3_ref[...], plus2_ref[...], (((1,), (1,)), ((), ())),
        preferred_element_type=jnp.float32) + b3_ref[...]


@jax.jit
def kernel(feat, feat1, feat2, inds, inds1, inds2, wei1, wei2,
           dens_feat_f, dens_feat_s, W1, b1, W3, b3):
    del wei2
    padp = PAD - NPTS

    def padt(x):
        return jnp.pad(x[0].T, ((0, padp), (0, 0)))

    def padflat(x, k):
        return jnp.pad(x[0].astype(jnp.int32).reshape(NPTS, k),
                       ((0, padp), (0, 0))).reshape(-1)

    ft = padt(feat)
    f1t = padt(feat1)
    f2t = padt(feat2)
    dfft = padt(dens_feat_f)
    dfst = padt(dens_feat_s)
    i8 = padflat(inds1, 8)
    i4 = padflat(inds, 4)
    i2 = padflat(inds2, 8)
    i14 = jnp.pad(inds1[0].astype(jnp.int32).reshape(NPTS, 8)[:, :4],
                  ((0, padp), (0, 0))).reshape(-1)
    wv = jnp.pad(wei1[0].reshape(NPTS, 8), ((0, padp), (0, 0))).reshape(-1)

    sc1, sc2 = _sc_kernels()
    plus_t = sc1(ft, f1t, f2t, i8, i4, wv)

    ds_t = pl.pallas_call(
        _tc_conv1_body,
        out_shape=jax.ShapeDtypeStruct((PAD, CF), jnp.float32),
    )(plus_t, W1.T, b1[None, :])

    plus2_t = sc2(ds_t, dfft, dfst, i2, i14)

    out = pl.pallas_call(
        _tc_conv2_body,
        out_shape=jax.ShapeDtypeStruct((CF, PAD), jnp.float32),
    )(plus2_t, W3, b3[:, None])
    return out[None, :, :NPTS]
